# baseline (speedup 1.0000x reference)
import functools

import jax
import jax.numpy as jnp
from jax import lax
from jax.experimental import pallas as pl
from jax.experimental.pallas import tpu as pltpu
from jax.experimental.pallas import tpu_sc as plsc

N = 1024
A = 128
K = 16
L = 16
NW = 32
JPW = N // NW
ACH = 32
NCH = A // ACH
IC = N // L
P = 32


def _sc_body(x_hbm, xt_hbm, adjt_hbm, out_hbm, adjrows, xchunk, stage):
    cid = lax.axis_index("c")
    sid = lax.axis_index("s")
    wid = sid * 2 + cid
    jbase = wid * JPW

    pltpu.sync_copy(adjt_hbm.at[pl.ds(jbase, JPW)], adjrows)

    ridx = 16 - lax.iota(jnp.int32, L)

    for ac in range(NCH):
        pltpu.sync_copy(xt_hbm.at[pl.ds(ac * ACH, ACH)], xchunk)
        pltpu.sync_copy(
            x_hbm.at[pl.ds(jbase, JPW), pl.ds(ac * ACH, ACH)],
            stage.at[:, 0, :],
        )

        def j_body(jl):
            jidx = jnp.full((L,), jl, jnp.int32)

            for g in range(ACH // P):
                def cbody(c, ts, g=g):
                    va = adjrows[jl, pl.ds(c * L, L)]
                    new_ts = []
                    for p in range(P):
                        vx = xchunk[g * P + p, pl.ds(c * L, L)]
                        vs = plsc.sort_key_val(va * vx, vx, descending=True)[0]
                        new_ts.append(jnp.sort(jnp.maximum(ts[p], vs)))
                    return tuple(new_ts)

                ts0 = tuple(
                    jnp.full((L,), -jnp.inf, jnp.float32) for _ in range(P)
                )
                ts = plsc.parallel_loop(0, IC, 1, unroll=1, carry=ts0)(cbody)
                for p in range(P):
                    cidx = jnp.full((L,), g * P + p, jnp.int32)
                    plsc.store_scatter(stage, [jidx, ridx, cidx], ts[p])

        plsc.parallel_loop(0, JPW, 1)(j_body)
        pltpu.sync_copy(
            stage, out_hbm.at[pl.ds(jbase, JPW), :, pl.ds(ac * ACH, ACH)]
        )


_sc_call = pl.kernel(
    _sc_body,
    out_type=jax.ShapeDtypeStruct((N, K + 1, A), jnp.float32),
    mesh=plsc.VectorSubcoreMesh(core_axis_name="c", subcore_axis_name="s"),
    compiler_params=pltpu.CompilerParams(
        use_tc_tiling_on_sc=False, needs_layout_passes=False
    ),
    scratch_types=[
        pltpu.VMEM((JPW, N), jnp.float32),
        pltpu.VMEM((ACH, N), jnp.float32),
        pltpu.VMEM((JPW, K + 1, ACH), jnp.float32),
    ],
)


@jax.jit
def kernel(x, adj):
    xt = x.T
    adjt = adj.T
    return _sc_call(x, xt, adjt)


# speedup vs baseline: 1.0208x; 1.0208x over previous
import functools

import jax
import jax.numpy as jnp
from jax import lax
from jax.experimental import pallas as pl
from jax.experimental.pallas import tpu as pltpu
from jax.experimental.pallas import tpu_sc as plsc

N = 1024
A = 128
K = 16
L = 16
NW = 32
JPW = N // NW
ACH = 32
NCH = A // ACH
IC = N // L
P = 32


def _sc_body(
    x_hbm, xt_hbm, adjt_hbm, out_hbm,
    adjrows, xch, stage, sem_a, sem_x0, sem_x1, sem_o,
):
    cid = lax.axis_index("c")
    sid = lax.axis_index("s")
    wid = sid * 2 + cid
    jbase = wid * JPW
    xsems = [sem_x0, sem_x1]

    cp_adj = pltpu.async_copy(adjt_hbm.at[pl.ds(jbase, JPW)], adjrows, sem_a)
    pltpu.async_copy(xt_hbm.at[pl.ds(0, ACH)], xch.at[0], sem_x0)
    cp_adj.wait()

    ridx = 16 - lax.iota(jnp.int32, L)

    for ac in range(NCH):
        cur = ac % 2
        pltpu.make_async_copy(
            xt_hbm.at[pl.ds(ac * ACH, ACH)], xch.at[cur], xsems[cur]
        ).wait()
        if ac + 1 < NCH:
            pltpu.async_copy(
                xt_hbm.at[pl.ds((ac + 1) * ACH, ACH)],
                xch.at[1 - cur],
                xsems[1 - cur],
            )
        if ac > 0:
            pltpu.make_async_copy(
                stage,
                out_hbm.at[pl.ds(jbase, JPW), :, pl.ds((ac - 1) * ACH, ACH)],
                sem_o,
            ).wait()
        pltpu.sync_copy(
            x_hbm.at[pl.ds(jbase, JPW), pl.ds(ac * ACH, ACH)],
            stage.at[:, 0, :],
        )

        def j_body(jl, cur=cur):
            jidx = jnp.full((L,), jl, jnp.int32)

            for g in range(ACH // P):
                def cbody(c, ts, g=g, cur=cur):
                    va = adjrows[jl, pl.ds(c * L, L)]
                    new_ts = []
                    for p in range(P):
                        vx = xch[cur, g * P + p, pl.ds(c * L, L)]
                        vs = plsc.sort_key_val(va * vx, vx, descending=True)[0]
                        new_ts.append(jnp.sort(jnp.maximum(ts[p], vs)))
                    return tuple(new_ts)

                ts0 = tuple(
                    jnp.full((L,), -jnp.inf, jnp.float32) for _ in range(P)
                )
                ts = plsc.parallel_loop(0, IC, 1, unroll=1, carry=ts0)(cbody)
                for p in range(P):
                    cidx = jnp.full((L,), g * P + p, jnp.int32)
                    plsc.store_scatter(stage, [jidx, ridx, cidx], ts[p])

        plsc.parallel_loop(0, JPW, 1)(j_body)
        pltpu.async_copy(
            stage, out_hbm.at[pl.ds(jbase, JPW), :, pl.ds(ac * ACH, ACH)], sem_o
        )

    pltpu.make_async_copy(
        stage,
        out_hbm.at[pl.ds(jbase, JPW), :, pl.ds((NCH - 1) * ACH, ACH)],
        sem_o,
    ).wait()


_sc_call = pl.kernel(
    _sc_body,
    out_type=jax.ShapeDtypeStruct((N, K + 1, A), jnp.float32),
    mesh=plsc.VectorSubcoreMesh(core_axis_name="c", subcore_axis_name="s"),
    compiler_params=pltpu.CompilerParams(
        use_tc_tiling_on_sc=False, needs_layout_passes=False
    ),
    scratch_types=[
        pltpu.VMEM((JPW, N), jnp.float32),
        pltpu.VMEM((2, ACH, N), jnp.float32),
        pltpu.VMEM((JPW, K + 1, ACH), jnp.float32),
        pltpu.SemaphoreType.DMA,
        pltpu.SemaphoreType.DMA,
        pltpu.SemaphoreType.DMA,
        pltpu.SemaphoreType.DMA,
    ],
)


@jax.jit
def kernel(x, adj):
    xt = x.T
    adjt = adj.T
    return _sc_call(x, xt, adjt)


# speedup vs baseline: 1.0410x; 1.0198x over previous
import jax
import jax.numpy as jnp
from jax import lax
from jax.experimental import pallas as pl
from jax.experimental.pallas import tpu as pltpu
from jax.experimental.pallas import tpu_sc as plsc

N = 1024
A = 128
K = 16
L = 16
NW = 32
JPW = N // NW
ACH = 32
NCH = A // ACH
IC = N // L
P = 32


def _sc_body(
    x_hbm, xt_hbm, adjt_hbm, out_hbm,
    adjrows, xch, stage, sem_a, sem_x0, sem_x1, sem_o,
):
    cid = lax.axis_index("c")
    sid = lax.axis_index("s")
    wid = sid * 2 + cid
    jbase = wid * JPW
    xsems = [sem_x0, sem_x1]

    cp_adj = pltpu.async_copy(adjt_hbm.at[pl.ds(jbase, JPW)], adjrows, sem_a)
    pltpu.async_copy(xt_hbm.at[pl.ds(0, ACH)], xch.at[0], sem_x0)
    cp_adj.wait()

    ridx = 16 - lax.iota(jnp.int32, L)

    for ac in range(NCH):
        cur = ac % 2
        pltpu.make_async_copy(
            xt_hbm.at[pl.ds(ac * ACH, ACH)], xch.at[cur], xsems[cur]
        ).wait()
        if ac + 1 < NCH:
            pltpu.async_copy(
                xt_hbm.at[pl.ds((ac + 1) * ACH, ACH)],
                xch.at[1 - cur],
                xsems[1 - cur],
            )
        if ac > 0:
            pltpu.make_async_copy(
                stage,
                out_hbm.at[pl.ds(jbase, JPW), :, pl.ds((ac - 1) * ACH, ACH)],
                sem_o,
            ).wait()
        cp_row0 = pltpu.async_copy(
            x_hbm.at[pl.ds(jbase, JPW), pl.ds(ac * ACH, ACH)],
            stage.at[:, 0, :],
            sem_a,
        )

        def j_body(jl, cur=cur):
            jidx = jnp.full((L,), jl, jnp.int32)

            for g in range(ACH // P):
                def cbody(c, ts, g=g, cur=cur):
                    va = adjrows[jl, pl.ds(c * L, L)]
                    new_ts = []
                    for p in range(P):
                        vx = xch[cur, g * P + p, pl.ds(c * L, L)]
                        vs = plsc.sort_key_val(va * vx, vx, descending=True)[0]
                        new_ts.append(jnp.sort(jnp.maximum(ts[p], vs)))
                    return tuple(new_ts)

                ts0 = tuple(
                    jnp.full((L,), -jnp.inf, jnp.float32) for _ in range(P)
                )
                ts = plsc.parallel_loop(0, IC, 1, unroll=1, carry=ts0)(cbody)
                for p in range(P):
                    cidx = jnp.full((L,), g * P + p, jnp.int32)
                    plsc.store_scatter(stage, [jidx, ridx, cidx], ts[p])

        plsc.parallel_loop(0, JPW, 1)(j_body)
        cp_row0.wait()
        pltpu.async_copy(
            stage, out_hbm.at[pl.ds(jbase, JPW), :, pl.ds(ac * ACH, ACH)], sem_o
        )

    pltpu.make_async_copy(
        stage,
        out_hbm.at[pl.ds(jbase, JPW), :, pl.ds((NCH - 1) * ACH, ACH)],
        sem_o,
    ).wait()


_sc_call = pl.kernel(
    _sc_body,
    out_type=jax.ShapeDtypeStruct((N, K + 1, A), jnp.float32),
    mesh=plsc.VectorSubcoreMesh(core_axis_name="c", subcore_axis_name="s"),
    compiler_params=pltpu.CompilerParams(
        use_tc_tiling_on_sc=False, needs_layout_passes=False
    ),
    scratch_types=[
        pltpu.VMEM((JPW, N), jnp.float32),
        pltpu.VMEM((2, ACH, N), jnp.float32),
        pltpu.VMEM((JPW, K + 1, ACH), jnp.float32),
        pltpu.SemaphoreType.DMA,
        pltpu.SemaphoreType.DMA,
        pltpu.SemaphoreType.DMA,
        pltpu.SemaphoreType.DMA,
    ],
)


@jax.jit
def kernel(x, adj):
    xt = x.T
    adjt = adj.T
    return _sc_call(x, xt, adjt)
